# baseline (speedup 1.0000x reference)
import jax
import jax.numpy as jnp
from jax.experimental import pallas as pl
from jax.experimental.pallas import tpu as pltpu

_LOG2E = 1.4426950408889634
_SCALE = (10.0 * _LOG2E) ** 0.5
_MCLIP = 20.0 * _LOG2E
_NEG = -1e30
_BR = 512
_BN = 4096
_PW = 128
_NC = 2


def _prep_kernel(x_ref, l_ref, o_ref, p_ref):
    x = x_ref[...]
    lab = l_ref[...]
    fg = lab >= 0
    bgf = jnp.where(fg, 0.0, 1.0)
    x2 = jnp.concatenate([x * x, bgf], axis=1)
    ones = jnp.concatenate(
        [jnp.ones((x.shape[1], 128), jnp.float32),
         jnp.full((1, 128), 1e30, jnp.float32)], axis=0)
    nrm2 = jax.lax.dot_general(x2, ones, (((1,), (0,)), ((), ())),
                               preferred_element_type=jnp.float32)
    scale = _SCALE * jax.lax.rsqrt(jnp.maximum(nrm2, 1e-24))
    o_ref[...] = (x * scale).astype(jnp.bfloat16)
    cid = jax.lax.broadcasted_iota(jnp.int32, p_ref.shape, 1)
    p = (cid == lab) | ((cid == _PW - 1) & fg)
    p_ref[...] = p.astype(jnp.bfloat16)


def _loss_kernel(fi_ref, f_ref, lr_ref, p_ref, ls_ref, cnt_ref):
    i = pl.program_id(0)
    br = fi_ref.shape[0]
    n = f_ref.shape[0] // 2
    ch = n // _NC
    fi = fi_ref[...]
    lrow = lr_ref[...]
    fg_row = lrow >= 0

    mx = jnp.full((br, 1), _NEG, jnp.float32)
    ep = jnp.zeros((br, _PW), jnp.float32)
    leye = (jax.lax.broadcasted_iota(jnp.int32, (br, br), 0)
            == jax.lax.broadcasted_iota(jnp.int32, (br, br), 1))
    for c in range(_NC):
        off = pl.multiple_of(i * br + c * ch, br)
        x = jax.lax.dot_general(fi, f_ref[pl.ds(off, ch), :],
                                (((1,), (1,)), ((), ())),
                                preferred_element_type=jnp.float32)
        if c == 0:
            x = jnp.concatenate(
                [jnp.where(leye, _NEG, x[:, :br]), x[:, br:]], axis=1)
        mx = jnp.maximum(mx, jnp.max(x, axis=1, keepdims=True))
        e = jnp.exp2(x).astype(jnp.bfloat16)
        ep = ep + jax.lax.dot_general(e, p_ref[pl.ds(off, ch), :],
                                      (((1,), (0,)), ((), ())),
                                      preferred_element_type=jnp.float32)
    m = jnp.clip(mx, -_MCLIP, _MCLIP)
    shift = jnp.exp2(-m)

    lane = jax.lax.broadcasted_iota(jnp.int32, (br, _PW), 1)
    pos_sum = jnp.sum(jnp.where(lane == lrow, ep, 0.0), axis=1,
                      keepdims=True)
    all_sum = jnp.sum(jnp.where(lane == _PW - 1, ep, 0.0), axis=1,
                      keepdims=True)

    pos_c = jnp.clip(pos_sum * shift, 1e-6, 1e6)
    all_c = jnp.clip(all_sum * shift, 1e-6, 1e6)
    loss = jnp.minimum(-jnp.log(pos_c / all_c), 10.0)

    valid = jnp.where(fg_row & (pos_sum > 0.0), 1.0, 0.0)
    ls_ref[...] = jnp.full(ls_ref.shape, jnp.sum(loss * valid), jnp.float32)
    cnt_ref[...] = jnp.full(cnt_ref.shape, jnp.sum(valid), jnp.float32)


def kernel(features, labels):
    n, d = features.shape
    labels = labels.astype(jnp.int32)
    nbn = n // _BN

    fn2, p2 = pl.pallas_call(
        _prep_kernel,
        out_shape=[jax.ShapeDtypeStruct((2 * n, d), jnp.bfloat16),
                   jax.ShapeDtypeStruct((2 * n, _PW), jnp.bfloat16)],
        grid=(2 * nbn,),
        in_specs=[pl.BlockSpec((_BN, d), lambda i: (i % nbn, 0)),
                  pl.BlockSpec((_BN, 1), lambda i: (i % nbn, 0))],
        out_specs=[pl.BlockSpec((_BN, d), lambda i: (i, 0)),
                   pl.BlockSpec((_BN, _PW), lambda i: (i, 0))],
        compiler_params=pltpu.CompilerParams(
            dimension_semantics=("parallel",)),
        name="cpe_prep",
    )(features, labels.reshape(n, 1))

    nb = n // _BR
    ls, cnt = pl.pallas_call(
        _loss_kernel,
        out_shape=[jax.ShapeDtypeStruct((nb, 1, 128), jnp.float32),
                   jax.ShapeDtypeStruct((nb, 1, 128), jnp.float32)],
        grid=(nb,),
        in_specs=[
            pl.BlockSpec((_BR, d), lambda i: (i, 0)),
            pl.BlockSpec((2 * n, d), lambda i: (0, 0)),
            pl.BlockSpec((_BR, 1), lambda i: (i, 0)),
            pl.BlockSpec((2 * n, _PW), lambda i: (0, 0)),
        ],
        out_specs=[pl.BlockSpec((1, 1, 128), lambda i: (i, 0, 0)),
                   pl.BlockSpec((1, 1, 128), lambda i: (i, 0, 0))],
        compiler_params=pltpu.CompilerParams(
            dimension_semantics=("parallel",),
            vmem_limit_bytes=56 * 1024 * 1024),
        name="cpe_loss",
    )(fn2, fn2, labels.reshape(n, 1), p2)

    total = jnp.sum(ls[:, 0, 0])
    n_valid = jnp.sum(cnt[:, 0, 0])
    mean = total / jnp.maximum(n_valid, 1.0)
    return jnp.where(n_valid > 0.0, mean, jnp.float32(0.0))


# speedup vs baseline: 1.3325x; 1.3325x over previous
import jax
import jax.numpy as jnp
from jax.experimental import pallas as pl
from jax.experimental.pallas import tpu as pltpu

_LOG2E = 1.4426950408889634
_SCALE = (10.0 * _LOG2E) ** 0.5
_MCLIP = 20.0 * _LOG2E
_NEG = -1e30
_BR = 512
_BN = 4096
_PW = 128
_NC = 1


def _prep_kernel(x_ref, l_ref, o_ref, p_ref):
    x = x_ref[...]
    lab = l_ref[...]
    fg = lab >= 0
    bgf = jnp.where(fg, 0.0, 1.0)
    x2 = jnp.concatenate([x * x, bgf], axis=1)
    ones = jnp.concatenate(
        [jnp.ones((x.shape[1], 128), jnp.float32),
         jnp.full((1, 128), 1e30, jnp.float32)], axis=0)
    nrm2 = jax.lax.dot_general(x2, ones, (((1,), (0,)), ((), ())),
                               preferred_element_type=jnp.float32)
    scale = _SCALE * jax.lax.rsqrt(jnp.maximum(nrm2, 1e-24))
    o_ref[...] = (x * scale).astype(jnp.bfloat16)
    cid = jax.lax.broadcasted_iota(jnp.int32, p_ref.shape, 1)
    p = (cid == lab) | ((cid == _PW - 1) & fg)
    p_ref[...] = p.astype(jnp.bfloat16)


def _loss_kernel(fi_ref, f_ref, lr_ref, p_ref, ls_ref, cnt_ref):
    i = pl.program_id(0)
    br = fi_ref.shape[0]
    n = f_ref.shape[0] // 2
    ch = n // _NC
    fi = fi_ref[...]
    lrow = lr_ref[...]
    fg_row = lrow >= 0

    mx = jnp.full((br, 1), _NEG, jnp.float32)
    ep = jnp.zeros((br, _PW), jnp.float32)
    leye = (jax.lax.broadcasted_iota(jnp.int32, (br, br), 0)
            == jax.lax.broadcasted_iota(jnp.int32, (br, br), 1))
    for c in range(_NC):
        off = pl.multiple_of(i * br + c * ch, br)
        x = jax.lax.dot_general(fi, f_ref[pl.ds(off, ch), :],
                                (((1,), (1,)), ((), ())),
                                preferred_element_type=jnp.float32)
        if c == 0:
            x = jnp.concatenate(
                [jnp.where(leye, _NEG, x[:, :br]), x[:, br:]], axis=1)
        mx = jnp.maximum(mx, jnp.max(x, axis=1, keepdims=True))
        e = jnp.exp2(x).astype(jnp.bfloat16)
        ep = ep + jax.lax.dot_general(e, p_ref[pl.ds(off, ch), :],
                                      (((1,), (0,)), ((), ())),
                                      preferred_element_type=jnp.float32)
    m = jnp.clip(mx, -_MCLIP, _MCLIP)
    shift = jnp.exp2(-m)

    lane = jax.lax.broadcasted_iota(jnp.int32, (br, _PW), 1)
    pos_sum = jnp.sum(jnp.where(lane == lrow, ep, 0.0), axis=1,
                      keepdims=True)
    all_sum = jnp.sum(jnp.where(lane == _PW - 1, ep, 0.0), axis=1,
                      keepdims=True)

    pos_c = jnp.clip(pos_sum * shift, 1e-6, 1e6)
    all_c = jnp.clip(all_sum * shift, 1e-6, 1e6)
    loss = jnp.minimum(-jnp.log(pos_c / all_c), 10.0)

    valid = jnp.where(fg_row & (pos_sum > 0.0), 1.0, 0.0)
    ls_ref[...] = jnp.full(ls_ref.shape, jnp.sum(loss * valid), jnp.float32)
    cnt_ref[...] = jnp.full(cnt_ref.shape, jnp.sum(valid), jnp.float32)


def kernel(features, labels):
    n, d = features.shape
    labels = labels.astype(jnp.int32)
    nbn = n // _BN

    fn2, p2 = pl.pallas_call(
        _prep_kernel,
        out_shape=[jax.ShapeDtypeStruct((2 * n, d), jnp.bfloat16),
                   jax.ShapeDtypeStruct((2 * n, _PW), jnp.bfloat16)],
        grid=(2 * nbn,),
        in_specs=[pl.BlockSpec((_BN, d), lambda i: (i % nbn, 0)),
                  pl.BlockSpec((_BN, 1), lambda i: (i % nbn, 0))],
        out_specs=[pl.BlockSpec((_BN, d), lambda i: (i, 0)),
                   pl.BlockSpec((_BN, _PW), lambda i: (i, 0))],
        compiler_params=pltpu.CompilerParams(
            dimension_semantics=("parallel",)),
        name="cpe_prep",
    )(features, labels.reshape(n, 1))

    nb = n // _BR
    ls, cnt = pl.pallas_call(
        _loss_kernel,
        out_shape=[jax.ShapeDtypeStruct((nb, 1, 128), jnp.float32),
                   jax.ShapeDtypeStruct((nb, 1, 128), jnp.float32)],
        grid=(nb,),
        in_specs=[
            pl.BlockSpec((_BR, d), lambda i: (i, 0)),
            pl.BlockSpec((2 * n, d), lambda i: (0, 0)),
            pl.BlockSpec((_BR, 1), lambda i: (i, 0)),
            pl.BlockSpec((2 * n, _PW), lambda i: (0, 0)),
        ],
        out_specs=[pl.BlockSpec((1, 1, 128), lambda i: (i, 0, 0)),
                   pl.BlockSpec((1, 1, 128), lambda i: (i, 0, 0))],
        compiler_params=pltpu.CompilerParams(
            dimension_semantics=("parallel",),
            vmem_limit_bytes=56 * 1024 * 1024),
        name="cpe_loss",
    )(fn2, fn2, labels.reshape(n, 1), p2)

    total = jnp.sum(ls[:, 0, 0])
    n_valid = jnp.sum(cnt[:, 0, 0])
    mean = total / jnp.maximum(n_valid, 1.0)
    return jnp.where(n_valid > 0.0, mean, jnp.float32(0.0))
